# baseline (speedup 1.0000x reference)
import functools

import jax
import jax.numpy as jnp
from jax import lax
from jax.experimental import pallas as pl
from jax.experimental.pallas import tpu as pltpu
from jax.experimental.pallas import tpu_sc as plsc

_N = 10000
_E = 320000
_D = 128
_DOUT = 64
_G = 64
_L = 64

_NC = 2
_NS = 16
_NW = _NC * _NS
_EPW = _E // _NW
_K = 80
_CH = _EPW // _K
_WIN = 25
_NWIN = _CH // _WIN
_KD = 125
_CHD = _EPW // _KD
_NP = 10240
_RPT = _NP // _NS
_ZR = 32


def _zero_zbuf(zbuf, width):
  zv = jnp.zeros((16,), jnp.float32)
  def body(i, _):
    r = i // (width // 16)
    l = i % (width // 16)
    zbuf[r, pl.ds(l * 16, 16)] = zv
    return 0
  lax.fori_loop(0, _ZR * (width // 16), body, 0)


def _zero_acc_slice(acc, zbuf, tile):
  def body(j, _):
    pltpu.sync_copy(zbuf, acc.at[pl.ds(tile * _RPT + j * _ZR, _ZR)])
    return 0
  lax.fori_loop(0, _RPT // _ZR, body, 0)


def _sc_deg_kernel(e3_hbm, ones_hbm, out_hbm, dst_v, ones_v, slice_v, acc,
                   sem):
  c = lax.axis_index("c")
  s = lax.axis_index("s")
  w = s * _NC + c
  pltpu.sync_copy(e3_hbm.at[1, w], dst_v)
  pltpu.sync_copy(ones_hbm, ones_v)
  zv = jnp.zeros((16,), jnp.float32)
  def zb(j, _):
    slice_v[pl.ds(j * 16, 16)] = zv
    return 0
  lax.fori_loop(0, _RPT // 16, zb, 0)
  pltpu.sync_copy(slice_v, acc.at[pl.ds(s * _RPT, _RPT)])
  plsc.subcore_barrier()
  def edge(i, _):
    pltpu.async_copy(ones_v, acc.at[dst_v.at[i]], sem, add=True)
    return 0
  lax.fori_loop(0, _CHD, edge, 0)
  def drain(i, _):
    pltpu.make_async_copy(ones_v, acc.at[dst_v.at[i]], sem).wait()
    return 0
  lax.fori_loop(0, _CHD, drain, 0)
  plsc.subcore_barrier()
  pltpu.sync_copy(acc.at[pl.ds(s * _RPT, _RPT)],
                  out_hbm.at[pl.ds(c * _NP + s * _RPT, _RPT)])


def _sc_agg_kernel(h_hbm, e4_hbm, out_hbm,
                   src_v, dst_v, rows0, rows1, rows2, rows3, acc,
                   sem0, sem1, sem2, sem3, ssem0, ssem1, ssem2, ssem3):
  c = lax.axis_index("c")
  s = lax.axis_index("s")
  w = s * _NC + c
  zv = jnp.zeros((16,), jnp.float32)
  def zb(i, _):
    rows0[i // 8, pl.ds((i % 8) * 16, 16)] = zv
    return 0
  lax.fori_loop(0, _K * 8, zb, 0)
  def za(j, _):
    pltpu.async_copy(rows0, acc.at[pl.ds(s * _RPT + j * _K, _K)], sem0)
    return 0
  lax.fori_loop(0, _RPT // _K, za, 0)
  def zd(j, _):
    pltpu.make_async_copy(rows0, acc.at[pl.ds(s * _RPT + j * _K, _K)],
                          sem0).wait()
    return 0
  lax.fori_loop(0, _RPT // _K, zd, 0)
  plsc.subcore_barrier()
  bufs = (rows0, rows1, rows2, rows3)
  sems = (sem0, sem1, sem2, sem3)
  ssems = (ssem0, ssem1, ssem2, ssem3)

  def wait_sc(t, b):
    pltpu.make_async_copy(bufs[b], acc.at[dst_v.at[t]], ssems[b]).wait()

  def step(t, b, do_issue, pre_wait):
    if do_issue:
      bi = (b + 3) % 4
      if pre_wait:
        wait_sc(t - 1, bi)
      pltpu.async_copy(h_hbm.at[src_v.at[t + 3]], bufs[bi], sems[bi])
    pltpu.make_async_copy(h_hbm.at[src_v.at[t]], bufs[b], sems[b]).wait()
    pltpu.async_copy(bufs[b], acc.at[dst_v.at[t]], ssems[b], add=True)

  def window(g, _):
    pltpu.sync_copy(e4_hbm.at[0, w, g], src_v)
    pltpu.sync_copy(e4_hbm.at[1, w, g], dst_v)
    pltpu.async_copy(h_hbm.at[src_v.at[0]], bufs[0], sems[0])
    pltpu.async_copy(h_hbm.at[src_v.at[1]], bufs[1], sems[1])
    pltpu.async_copy(h_hbm.at[src_v.at[2]], bufs[2], sems[2])
    step(0, 0, True, False)
    def body(j, _):
      base = 4 * j + 1
      for k in range(4):
        step(base + k, (1 + k) % 4, True, True)
      return 0
    lax.fori_loop(0, (_WIN - 5) // 4, body, 0)
    for t in range(_WIN - 4, _WIN):
      step(t, t % 4, t + 3 < _WIN, t + 3 < _WIN)
    for t in range(_WIN - 4, _WIN):
      wait_sc(t, t % 4)
    return 0
  lax.fori_loop(0, _NWIN, window, 0)
  plsc.subcore_barrier()
  pltpu.sync_copy(acc.at[pl.ds(s * _RPT, _RPT)],
                  out_hbm.at[c, pl.ds(s * _RPT, _RPT)])


@functools.cache
def _sc_calls():
  mesh = plsc.VectorSubcoreMesh(core_axis_name="c", subcore_axis_name="s",
                                num_cores=_NC, num_subcores=_NS)
  deg_call = pl.kernel(
      _sc_deg_kernel,
      out_type=jax.ShapeDtypeStruct((_NC * _NP,), jnp.float32),
      mesh=mesh,
      scratch_types=[
          pltpu.VMEM((_CHD, _KD), jnp.int32),
          pltpu.VMEM((_KD,), jnp.float32),
          pltpu.VMEM((_RPT,), jnp.float32),
          pltpu.VMEM_SHARED((_NP,), jnp.float32),
          pltpu.SemaphoreType.DMA,
      ],
  )
  agg_call = pl.kernel(
      _sc_agg_kernel,
      out_type=jax.ShapeDtypeStruct((_NC, _NP, _D), jnp.float32),
      mesh=mesh,
      scratch_types=[
          pltpu.VMEM((_WIN, _K), jnp.int32),
          pltpu.VMEM((_WIN, _K), jnp.int32),
          pltpu.VMEM((_K, _D), jnp.float32),
          pltpu.VMEM((_K, _D), jnp.float32),
          pltpu.VMEM((_K, _D), jnp.float32),
          pltpu.VMEM((_K, _D), jnp.float32),
          pltpu.VMEM_SHARED((_NP, _D), jnp.float32),
          pltpu.SemaphoreType.DMA,
          pltpu.SemaphoreType.DMA,
          pltpu.SemaphoreType.DMA,
          pltpu.SemaphoreType.DMA,
          pltpu.SemaphoreType.DMA,
          pltpu.SemaphoreType.DMA,
          pltpu.SemaphoreType.DMA,
          pltpu.SemaphoreType.DMA,
      ],
  )
  return deg_call, agg_call


_BN = 10000
_NB = _N // _BN


def _tc_b_kernel(x_ref, w1_ref, d0_ref, d1_ref, h1p_ref, dinv_ref):
  deg = d0_ref[0] + d1_ref[0] + 1.0
  dinv = lax.rsqrt(jnp.maximum(deg, 1.0))
  z1 = jnp.dot(x_ref[...], w1_ref[...], preferred_element_type=jnp.float32)
  h1p_ref[...] = z1 * dinv
  dinv_ref[...] = dinv


def _tc_d_kernel(p0_ref, p1_ref, h1p_ref, dinv_ref, b1_ref, w2_ref,
                 h2p_ref):
  dinv = dinv_ref[...]
  h1 = jnp.maximum(
      dinv * (p0_ref[0] + p1_ref[0] + h1p_ref[...]) + b1_ref[...], 0.0)
  z2 = jnp.dot(h1, w2_ref[...], preferred_element_type=jnp.float32)
  h2p_ref[...] = z2 * dinv


def _tc_f_kernel(p0_ref, p1_ref, h2p_ref, dinv_ref, b2_ref, wp_ref, bp_ref,
                 batch_ref, nli_ref, out_ref, acc_ref):
  i = pl.program_id(0)

  @pl.when(i == 0)
  def _():
    acc_ref[...] = jnp.zeros((_G, _D), jnp.float32)

  dinv = dinv_ref[...]
  h2 = jnp.maximum(
      dinv * (p0_ref[0] + p1_ref[0] + h2p_ref[...]) + b2_ref[...], 0.0)
  gids = lax.broadcasted_iota(jnp.int32, (_BN, _G), 1)
  onehot = (batch_ref[...] == gids).astype(jnp.float32)
  acc_ref[...] += lax.dot_general(
      onehot, h2, (((0,), (0,)), ((), ())),
      preferred_element_type=jnp.float32)

  @pl.when(i == _NB - 1)
  def _():
    pooled = acc_ref[...]
    feat = jnp.dot(pooled, wp_ref[...],
                   preferred_element_type=jnp.float32) + bp_ref[...]
    lids = lax.broadcasted_iota(jnp.int32, (_L, _G), 1)
    sel = (nli_ref[...] == lids).astype(jnp.float32)
    pred = jnp.dot(sel, feat, preferred_element_type=jnp.float32)
    m = jnp.max(pred, axis=1, keepdims=True)
    ex = jnp.exp(pred - m)
    lse = jnp.log(jnp.sum(ex, axis=1, keepdims=True)) + m
    out_ref[...] = pred - lse


def _row_block(width):
  return pl.BlockSpec((_BN, width), lambda i: (i, 0))


def _part_block(width, core):
  return pl.BlockSpec((1, _BN, width), lambda i, c=core: (c, i, 0))


def _full(shape):
  return pl.BlockSpec(shape, lambda i: tuple(0 for _ in shape))


_tc_b_call = pl.pallas_call(
    _tc_b_kernel,
    grid=(_NB,),
    in_specs=[_row_block(_D), _full((_D, _D)), _part_block(1, 0),
              _part_block(1, 1)],
    out_specs=[_row_block(_D), _row_block(1)],
    out_shape=[
        jax.ShapeDtypeStruct((_N, _D), jnp.float32),
        jax.ShapeDtypeStruct((_N, 1), jnp.float32),
    ],
)

_tc_d_call = pl.pallas_call(
    _tc_d_kernel,
    grid=(_NB,),
    in_specs=[_part_block(_D, 0), _part_block(_D, 1), _row_block(_D),
              _row_block(1), _full((1, _D)), _full((_D, _D))],
    out_specs=_row_block(_D),
    out_shape=jax.ShapeDtypeStruct((_N, _D), jnp.float32),
)

_tc_f_call = pl.pallas_call(
    _tc_f_kernel,
    grid=(_NB,),
    in_specs=[_part_block(_D, 0), _part_block(_D, 1), _row_block(_D),
              _row_block(1), _full((1, _D)), _full((_D, _DOUT)),
              _full((1, _DOUT)), _row_block(1), _full((_L, 1))],
    out_specs=_full((_L, _DOUT)),
    out_shape=jax.ShapeDtypeStruct((_L, _DOUT), jnp.float32),
    scratch_shapes=[pltpu.VMEM((_G, _D), jnp.float32)],
    compiler_params=pltpu.CompilerParams(
        dimension_semantics=("arbitrary",)),
)


@jax.jit
def kernel(x, edge_index, batch_vec, node_label_index, node_label,
           W1, b1, W2, b2, Wp, bp):
  deg_call, agg_call = _sc_calls()
  e3 = edge_index.reshape(2, _NW, _CHD, _KD)
  e4 = edge_index.reshape(2, _NW, _NWIN, _WIN, _K)

  degf = deg_call(e3, jnp.ones((_KD,), jnp.float32))
  degp = degf.reshape(_NC, _NP, 1)
  h1p, dinv = _tc_b_call(x, W1, degp, degp)

  p1 = agg_call(h1p, e4)
  h2p = _tc_d_call(p1, p1, h1p, dinv, b1.reshape(1, _D), W2)

  p2 = agg_call(h2p, e4)
  out = _tc_f_call(p2, p2, h2p, dinv, b2.reshape(1, _D),
                   Wp, bp.reshape(1, _DOUT),
                   batch_vec.reshape(_N, 1),
                   node_label_index.reshape(_L, 1))
  return (out, node_label)


# speedup vs baseline: 1.0163x; 1.0163x over previous
import functools

import jax
import jax.numpy as jnp
from jax import lax
from jax.experimental import pallas as pl
from jax.experimental.pallas import tpu as pltpu
from jax.experimental.pallas import tpu_sc as plsc

_N = 10000
_E = 320000
_D = 128
_DOUT = 64
_G = 64
_L = 64

_NC = 2
_NS = 16
_NW = _NC * _NS
_EPW = _E // _NW
_K = 80
_CH = _EPW // _K
_WIN = 25
_NWIN = _CH // _WIN
_KD = 125
_CHD = _EPW // _KD
_NP = 10240
_RPT = _NP // _NS
_ZR = 32


def _zero_zbuf(zbuf, width):
  zv = jnp.zeros((16,), jnp.float32)
  def body(i, _):
    r = i // (width // 16)
    l = i % (width // 16)
    zbuf[r, pl.ds(l * 16, 16)] = zv
    return 0
  lax.fori_loop(0, _ZR * (width // 16), body, 0)


def _zero_acc_slice(acc, zbuf, tile):
  def body(j, _):
    pltpu.sync_copy(zbuf, acc.at[pl.ds(tile * _RPT + j * _ZR, _ZR)])
    return 0
  lax.fori_loop(0, _RPT // _ZR, body, 0)


def _sc_deg_kernel(e3_hbm, ones_hbm, out_hbm, dst_v, ones_v, slice_v, acc,
                   sem):
  c = lax.axis_index("c")
  s = lax.axis_index("s")
  w = s * _NC + c
  pltpu.sync_copy(e3_hbm.at[1, w], dst_v)
  pltpu.sync_copy(ones_hbm, ones_v)
  zv = jnp.zeros((16,), jnp.float32)
  def zb(j, _):
    slice_v[pl.ds(j * 16, 16)] = zv
    return 0
  lax.fori_loop(0, _RPT // 16, zb, 0)
  pltpu.sync_copy(slice_v, acc.at[pl.ds(s * _RPT, _RPT)])
  plsc.subcore_barrier()
  def edge(i, _):
    pltpu.async_copy(ones_v, acc.at[dst_v.at[i]], sem, add=True)
    return 0
  lax.fori_loop(0, _CHD, edge, 0)
  def drain(i, _):
    pltpu.make_async_copy(ones_v, acc.at[dst_v.at[i]], sem).wait()
    return 0
  lax.fori_loop(0, _CHD, drain, 0)
  plsc.subcore_barrier()
  pltpu.sync_copy(acc.at[pl.ds(s * _RPT, _RPT)],
                  out_hbm.at[pl.ds(c * _NP + s * _RPT, _RPT)])


def _sc_agg_kernel(h_hbm, e4_hbm, out_hbm,
                   src_v, dst_v, rows0, rows1, rows2, rows3, acc,
                   sem0, sem1, sem2, sem3, ssem0, ssem1, ssem2, ssem3):
  c = lax.axis_index("c")
  s = lax.axis_index("s")
  w = s * _NC + c
  zv = jnp.zeros((16,), jnp.float32)
  def zb(i, _):
    rows0[i // 8, pl.ds((i % 8) * 16, 16)] = zv
    return 0
  lax.fori_loop(0, _K * 8, zb, 0)
  def za(j, _):
    pltpu.async_copy(rows0, acc.at[pl.ds(s * _RPT + j * _K, _K)], sem0)
    return 0
  lax.fori_loop(0, _RPT // _K, za, 0)
  def zd(j, _):
    pltpu.make_async_copy(rows0, acc.at[pl.ds(s * _RPT + j * _K, _K)],
                          sem0).wait()
    return 0
  lax.fori_loop(0, _RPT // _K, zd, 0)
  plsc.subcore_barrier()
  bufs = (rows0, rows1, rows2, rows3)
  sems = (sem0, sem1, sem2, sem3)
  ssems = (ssem0, ssem1, ssem2, ssem3)

  def wait_sc(t, b):
    pltpu.make_async_copy(bufs[b], acc.at[dst_v.at[t]], ssems[b]).wait()

  def step(t, b, do_issue, pre_wait):
    if do_issue:
      bi = (b + 3) % 4
      if pre_wait:
        wait_sc(t - 1, bi)
      pltpu.async_copy(h_hbm.at[src_v.at[t + 3]], bufs[bi], sems[bi])
    pltpu.make_async_copy(h_hbm.at[src_v.at[t]], bufs[b], sems[b]).wait()
    pltpu.async_copy(bufs[b], acc.at[dst_v.at[t]], ssems[b], add=True)

  def window(g, _):
    pltpu.sync_copy(e4_hbm.at[0, w, g], src_v)
    pltpu.sync_copy(e4_hbm.at[1, w, g], dst_v)
    pltpu.async_copy(h_hbm.at[src_v.at[0]], bufs[0], sems[0])
    pltpu.async_copy(h_hbm.at[src_v.at[1]], bufs[1], sems[1])
    pltpu.async_copy(h_hbm.at[src_v.at[2]], bufs[2], sems[2])
    step(0, 0, True, False)
    def body(j, _):
      base = 4 * j + 1
      for k in range(4):
        step(base + k, (1 + k) % 4, True, True)
      return 0
    lax.fori_loop(0, (_WIN - 5) // 4, body, 0)
    for t in range(_WIN - 4, _WIN):
      step(t, t % 4, t + 3 < _WIN, t + 3 < _WIN)
    for t in range(_WIN - 4, _WIN):
      wait_sc(t, t % 4)
    return 0
  lax.fori_loop(0, _NWIN, window, 0)
  plsc.subcore_barrier()
  pltpu.sync_copy(acc.at[pl.ds(s * _RPT, _RPT)],
                  out_hbm.at[c, pl.ds(s * _RPT, _RPT)])


@functools.cache
def _sc_calls():
  mesh = plsc.VectorSubcoreMesh(core_axis_name="c", subcore_axis_name="s",
                                num_cores=_NC, num_subcores=_NS)
  deg_call = pl.kernel(
      _sc_deg_kernel,
      out_type=jax.ShapeDtypeStruct((_NC * _NP,), jnp.float32),
      mesh=mesh,
      scratch_types=[
          pltpu.VMEM((_CHD, _KD), jnp.int32),
          pltpu.VMEM((_KD,), jnp.float32),
          pltpu.VMEM((_RPT,), jnp.float32),
          pltpu.VMEM_SHARED((_NP,), jnp.float32),
          pltpu.SemaphoreType.DMA,
      ],
  )
  agg_call = pl.kernel(
      _sc_agg_kernel,
      out_type=jax.ShapeDtypeStruct((_NC, _NP, _D), jnp.float32),
      mesh=mesh,
      scratch_types=[
          pltpu.VMEM((_WIN, _K), jnp.int32),
          pltpu.VMEM((_WIN, _K), jnp.int32),
          pltpu.VMEM((_K, _D), jnp.float32),
          pltpu.VMEM((_K, _D), jnp.float32),
          pltpu.VMEM((_K, _D), jnp.float32),
          pltpu.VMEM((_K, _D), jnp.float32),
          pltpu.VMEM_SHARED((_NP, _D), jnp.float32),
          pltpu.SemaphoreType.DMA,
          pltpu.SemaphoreType.DMA,
          pltpu.SemaphoreType.DMA,
          pltpu.SemaphoreType.DMA,
          pltpu.SemaphoreType.DMA,
          pltpu.SemaphoreType.DMA,
          pltpu.SemaphoreType.DMA,
          pltpu.SemaphoreType.DMA,
      ],
  )
  return deg_call, agg_call


_BN = 5000
_NB = _N // _BN


def _tc_b_kernel(x_ref, w1_ref, d0_ref, d1_ref, h1p_ref, dinv_ref):
  deg = d0_ref[0] + d1_ref[0] + 1.0
  dinv = lax.rsqrt(jnp.maximum(deg, 1.0))
  z1 = jnp.dot(x_ref[...], w1_ref[...], preferred_element_type=jnp.float32)
  h1p_ref[...] = z1 * dinv
  dinv_ref[...] = dinv


def _tc_d_kernel(p0_ref, p1_ref, h1p_ref, dinv_ref, b1_ref, w2_ref,
                 h2p_ref):
  dinv = dinv_ref[...]
  h1 = jnp.maximum(
      dinv * (p0_ref[0] + p1_ref[0] + h1p_ref[...]) + b1_ref[...], 0.0)
  z2 = jnp.dot(h1, w2_ref[...], preferred_element_type=jnp.float32)
  h2p_ref[...] = z2 * dinv


def _tc_f_kernel(p0_ref, p1_ref, h2p_ref, dinv_ref, b2_ref, wp_ref, bp_ref,
                 batch_ref, nli_ref, out_ref, acc_ref):
  i = pl.program_id(0)

  @pl.when(i == 0)
  def _():
    acc_ref[...] = jnp.zeros((_G, _D), jnp.float32)

  dinv = dinv_ref[...]
  h2 = jnp.maximum(
      dinv * (p0_ref[0] + p1_ref[0] + h2p_ref[...]) + b2_ref[...], 0.0)
  gids = lax.broadcasted_iota(jnp.int32, (_BN, _G), 1)
  onehot = (batch_ref[...] == gids).astype(jnp.float32)
  acc_ref[...] += lax.dot_general(
      onehot, h2, (((0,), (0,)), ((), ())),
      preferred_element_type=jnp.float32)

  @pl.when(i == _NB - 1)
  def _():
    pooled = acc_ref[...]
    feat = jnp.dot(pooled, wp_ref[...],
                   preferred_element_type=jnp.float32) + bp_ref[...]
    lids = lax.broadcasted_iota(jnp.int32, (_L, _G), 1)
    sel = (nli_ref[...] == lids).astype(jnp.float32)
    pred = jnp.dot(sel, feat, preferred_element_type=jnp.float32)
    m = jnp.max(pred, axis=1, keepdims=True)
    ex = jnp.exp(pred - m)
    lse = jnp.log(jnp.sum(ex, axis=1, keepdims=True)) + m
    out_ref[...] = pred - lse


def _row_block(width):
  return pl.BlockSpec((_BN, width), lambda i: (i, 0))


def _part_block(width, core):
  return pl.BlockSpec((1, _BN, width), lambda i, c=core: (c, i, 0))


def _full(shape):
  return pl.BlockSpec(shape, lambda i: tuple(0 for _ in shape))


_tc_b_call = pl.pallas_call(
    _tc_b_kernel,
    grid=(_NB,),
    in_specs=[_row_block(_D), _full((_D, _D)), _part_block(1, 0),
              _part_block(1, 1)],
    out_specs=[_row_block(_D), _row_block(1)],
    out_shape=[
        jax.ShapeDtypeStruct((_N, _D), jnp.float32),
        jax.ShapeDtypeStruct((_N, 1), jnp.float32),
    ],
)

_tc_d_call = pl.pallas_call(
    _tc_d_kernel,
    grid=(_NB,),
    in_specs=[_part_block(_D, 0), _part_block(_D, 1), _row_block(_D),
              _row_block(1), _full((1, _D)), _full((_D, _D))],
    out_specs=_row_block(_D),
    out_shape=jax.ShapeDtypeStruct((_N, _D), jnp.float32),
)

_tc_f_call = pl.pallas_call(
    _tc_f_kernel,
    grid=(_NB,),
    in_specs=[_part_block(_D, 0), _part_block(_D, 1), _row_block(_D),
              _row_block(1), _full((1, _D)), _full((_D, _DOUT)),
              _full((1, _DOUT)), _row_block(1), _full((_L, 1))],
    out_specs=_full((_L, _DOUT)),
    out_shape=jax.ShapeDtypeStruct((_L, _DOUT), jnp.float32),
    scratch_shapes=[pltpu.VMEM((_G, _D), jnp.float32)],
    compiler_params=pltpu.CompilerParams(
        dimension_semantics=("arbitrary",)),
)


@jax.jit
def kernel(x, edge_index, batch_vec, node_label_index, node_label,
           W1, b1, W2, b2, Wp, bp):
  deg_call, agg_call = _sc_calls()
  e3 = edge_index.reshape(2, _NW, _CHD, _KD)
  e4 = edge_index.reshape(2, _NW, _NWIN, _WIN, _K)

  degf = deg_call(e3, jnp.ones((_KD,), jnp.float32))
  degp = degf.reshape(_NC, _NP, 1)
  h1p, dinv = _tc_b_call(x, W1, degp, degp)

  p1 = agg_call(h1p, e4)
  h2p = _tc_d_call(p1, p1, h1p, dinv, b1.reshape(1, _D), W2)

  p2 = agg_call(h2p, e4)
  out = _tc_f_call(p2, p2, h2p, dinv, b2.reshape(1, _D),
                   Wp, bp.reshape(1, _DOUT),
                   batch_vec.reshape(_N, 1),
                   node_label_index.reshape(_L, 1))
  return (out, node_label)
